# baseline (speedup 1.0000x reference)
import functools

import jax
import jax.numpy as jnp
from jax.experimental import pallas as pl
from jax.experimental.pallas import tpu as pltpu


def _rgcn_body(x_k_ref, x_i_ref, a0_ref, a1_ref, ws_ref, w0_ref, w1_ref,
               b_ref, o_ref, y0_s, y1_s, acc_ref):
    i = pl.program_id(0)
    k = pl.program_id(1)
    nk = pl.num_programs(1)

    @pl.when(i == 0)
    def _compute_y():
        xk = x_k_ref[...]
        y0_s[k] = jnp.dot(xk, w0_ref[...],
                          preferred_element_type=jnp.float32).astype(jnp.bfloat16)
        y1_s[k] = jnp.dot(xk, w1_ref[...],
                          preferred_element_type=jnp.float32).astype(jnp.bfloat16)

    @pl.when(k == 0)
    def _init_acc():
        acc_ref[...] = jnp.dot(x_i_ref[...], ws_ref[...],
                               preferred_element_type=jnp.float32) + b_ref[...]

    a0 = a0_ref[...].astype(jnp.bfloat16)
    a1 = a1_ref[...].astype(jnp.bfloat16)
    acc_ref[...] += (
        jnp.dot(a0, y0_s[k], preferred_element_type=jnp.float32)
        + jnp.dot(a1, y1_s[k], preferred_element_type=jnp.float32))

    @pl.when(k == nk - 1)
    def _epilogue():
        o_ref[...] = jnp.maximum(acc_ref[...], 0.0)


@functools.partial(jax.jit, static_argnames=("tm", "tk"))
def _rgcn(x, a0, a1, ws, w0, w1, b, tm=512, tk=1024):
    n, f = x.shape
    u = ws.shape[1]
    ni = n // tm
    nk = n // tk
    b2 = b.reshape(1, u)
    out = pl.pallas_call(
        _rgcn_body,
        grid=(ni, nk),
        in_specs=[
            pl.BlockSpec((tk, f), lambda i, k: (jnp.where(i == 0, k, 0), 0)),
            pl.BlockSpec((tm, f), lambda i, k: (i, 0)),
            pl.BlockSpec((tm, tk), lambda i, k: (i, k)),
            pl.BlockSpec((tm, tk), lambda i, k: (i, k)),
            pl.BlockSpec((f, u), lambda i, k: (0, 0)),
            pl.BlockSpec((f, u), lambda i, k: (0, 0)),
            pl.BlockSpec((f, u), lambda i, k: (0, 0)),
            pl.BlockSpec((1, u), lambda i, k: (0, 0)),
        ],
        out_specs=pl.BlockSpec((tm, u), lambda i, k: (i, 0)),
        out_shape=jax.ShapeDtypeStruct((n, u), jnp.float32),
        scratch_shapes=[
            pltpu.VMEM((nk, tk, u), jnp.bfloat16),
            pltpu.VMEM((nk, tk, u), jnp.bfloat16),
            pltpu.VMEM((tm, u), jnp.float32),
        ],
        compiler_params=pltpu.CompilerParams(
            dimension_semantics=("arbitrary", "arbitrary")),
    )(x, x, a0, a1, ws, w0, w1, b2)
    return out


def kernel(features, A_0, A_1, self_kernel, rel_kernel_0, rel_kernel_1, bias):
    x = features[0]
    out = _rgcn(x, A_0, A_1, self_kernel, rel_kernel_0, rel_kernel_1, bias)
    return out[None, ...]


# speedup vs baseline: 1.0061x; 1.0061x over previous
import functools

import jax
import jax.numpy as jnp
from jax.experimental import pallas as pl
from jax.experimental.pallas import tpu as pltpu


def _rgcn_body(x_k_ref, x_i_ref, a0_ref, a1_ref, ws_ref, w0_ref, w1_ref,
               b_ref, o_ref, y0_s, y1_s, acc_ref):
    i = pl.program_id(0)
    k = pl.program_id(1)
    nk = pl.num_programs(1)

    @pl.when(i == 0)
    def _compute_y():
        xk = x_k_ref[...]
        y0_s[k] = jnp.dot(xk, w0_ref[...], preferred_element_type=jnp.float32)
        y1_s[k] = jnp.dot(xk, w1_ref[...], preferred_element_type=jnp.float32)

    @pl.when(k == 0)
    def _init_acc():
        acc_ref[...] = jnp.dot(x_i_ref[...], ws_ref[...],
                               preferred_element_type=jnp.float32) + b_ref[...]

    acc_ref[...] += (
        jnp.dot(a0_ref[...], y0_s[k], preferred_element_type=jnp.float32)
        + jnp.dot(a1_ref[...], y1_s[k], preferred_element_type=jnp.float32))

    @pl.when(k == nk - 1)
    def _epilogue():
        o_ref[...] = jnp.maximum(acc_ref[...], 0.0)


@functools.partial(jax.jit, static_argnames=("tm", "tk"))
def _rgcn(x, a0, a1, ws, w0, w1, b, tm=512, tk=1024):
    n, f = x.shape
    u = ws.shape[1]
    ni = n // tm
    nk = n // tk
    b2 = b.reshape(1, u)
    out = pl.pallas_call(
        _rgcn_body,
        grid=(ni, nk),
        in_specs=[
            pl.BlockSpec((tk, f), lambda i, k: (jnp.where(i == 0, k, 0), 0)),
            pl.BlockSpec((tm, f), lambda i, k: (i, 0)),
            pl.BlockSpec((tm, tk), lambda i, k: (i, k)),
            pl.BlockSpec((tm, tk), lambda i, k: (i, k)),
            pl.BlockSpec((f, u), lambda i, k: (0, 0)),
            pl.BlockSpec((f, u), lambda i, k: (0, 0)),
            pl.BlockSpec((f, u), lambda i, k: (0, 0)),
            pl.BlockSpec((1, u), lambda i, k: (0, 0)),
        ],
        out_specs=pl.BlockSpec((tm, u), lambda i, k: (i, 0)),
        out_shape=jax.ShapeDtypeStruct((n, u), jnp.float32),
        scratch_shapes=[
            pltpu.VMEM((nk, tk, u), jnp.float32),
            pltpu.VMEM((nk, tk, u), jnp.float32),
            pltpu.VMEM((tm, u), jnp.float32),
        ],
        compiler_params=pltpu.CompilerParams(
            dimension_semantics=("arbitrary", "arbitrary")),
    )(x, x, a0, a1, ws, w0, w1, b2)
    return out


def kernel(features, A_0, A_1, self_kernel, rel_kernel_0, rel_kernel_1, bias):
    x = features[0]
    out = _rgcn(x, A_0, A_1, self_kernel, rel_kernel_0, rel_kernel_1, bias)
    return out[None, ...]


# speedup vs baseline: 1.2428x; 1.2353x over previous
import functools

import jax
import jax.numpy as jnp
from jax.experimental import pallas as pl
from jax.experimental.pallas import tpu as pltpu


def _rgcn_body(x_k_ref, x_i_ref, a0_ref, a1_ref, ws_ref, w0_ref, w1_ref,
               b_ref, o_ref, y0_s, y1_s, acc_ref):
    i = pl.program_id(0)
    k = pl.program_id(1)
    nk = pl.num_programs(1)

    @pl.when(i == 0)
    def _compute_y():
        xk = x_k_ref[...]
        y0_s[k] = jnp.dot(xk, w0_ref[...], preferred_element_type=jnp.float32)
        y1_s[k] = jnp.dot(xk, w1_ref[...], preferred_element_type=jnp.float32)

    @pl.when(k == 0)
    def _init_acc():
        acc_ref[...] = jnp.dot(x_i_ref[...], ws_ref[...],
                               preferred_element_type=jnp.float32) + b_ref[...]

    acc_ref[...] += (
        jnp.dot(a0_ref[...], y0_s[k], preferred_element_type=jnp.float32)
        + jnp.dot(a1_ref[...], y1_s[k], preferred_element_type=jnp.float32))

    @pl.when(k == nk - 1)
    def _epilogue():
        o_ref[...] = jnp.maximum(acc_ref[...], 0.0)


@functools.partial(jax.jit, static_argnames=("tm", "tk"))
def _rgcn(x, a0, a1, ws, w0, w1, b, tm=128, tk=8192):
    n, f = x.shape
    u = ws.shape[1]
    ni = n // tm
    nk = n // tk
    b2 = b.reshape(1, u)
    out = pl.pallas_call(
        _rgcn_body,
        grid=(ni, nk),
        in_specs=[
            pl.BlockSpec((tk, f), lambda i, k: (jnp.where(i == 0, k, 0), 0)),
            pl.BlockSpec((tm, f), lambda i, k: (i, 0)),
            pl.BlockSpec((tm, tk), lambda i, k: (i, k)),
            pl.BlockSpec((tm, tk), lambda i, k: (i, k)),
            pl.BlockSpec((f, u), lambda i, k: (0, 0)),
            pl.BlockSpec((f, u), lambda i, k: (0, 0)),
            pl.BlockSpec((f, u), lambda i, k: (0, 0)),
            pl.BlockSpec((1, u), lambda i, k: (0, 0)),
        ],
        out_specs=pl.BlockSpec((tm, u), lambda i, k: (i, 0)),
        out_shape=jax.ShapeDtypeStruct((n, u), jnp.float32),
        scratch_shapes=[
            pltpu.VMEM((nk, tk, u), jnp.float32),
            pltpu.VMEM((nk, tk, u), jnp.float32),
            pltpu.VMEM((tm, u), jnp.float32),
        ],
        compiler_params=pltpu.CompilerParams(
            dimension_semantics=("arbitrary", "arbitrary")),
    )(x, x, a0, a1, ws, w0, w1, b2)
    return out


def kernel(features, A_0, A_1, self_kernel, rel_kernel_0, rel_kernel_1, bias):
    x = features[0]
    out = _rgcn(x, A_0, A_1, self_kernel, rel_kernel_0, rel_kernel_1, bias)
    return out[None, ...]


# speedup vs baseline: 1.2511x; 1.0067x over previous
import functools

import jax
import jax.numpy as jnp
from jax.experimental import pallas as pl
from jax.experimental.pallas import tpu as pltpu


def _rgcn_body(x_k_ref, x_i_ref, a0_ref, a1_ref, ws_ref, w0_ref, w1_ref,
               b_ref, o_ref, y0_s, y1_s, acc_ref):
    i = pl.program_id(0)
    k = pl.program_id(1)
    nk = pl.num_programs(1)

    @pl.when(i == 0)
    def _compute_y():
        xk = x_k_ref[...]
        y0_s[k] = jnp.dot(xk, w0_ref[...], preferred_element_type=jnp.float32)
        y1_s[k] = jnp.dot(xk, w1_ref[...], preferred_element_type=jnp.float32)

    @pl.when(k == 0)
    def _init_acc():
        acc_ref[...] = jnp.dot(x_i_ref[...], ws_ref[...],
                               preferred_element_type=jnp.float32) + b_ref[...]

    acc_ref[...] += (
        jnp.dot(a0_ref[...], y0_s[k], preferred_element_type=jnp.float32)
        + jnp.dot(a1_ref[...], y1_s[k], preferred_element_type=jnp.float32))

    @pl.when(k == nk - 1)
    def _epilogue():
        o_ref[...] = jnp.maximum(acc_ref[...], 0.0)


@functools.partial(jax.jit, static_argnames=("tm", "tk"))
def _rgcn(x, a0, a1, ws, w0, w1, b, tm=256, tk=8192):
    n, f = x.shape
    u = ws.shape[1]
    ni = n // tm
    nk = n // tk
    b2 = b.reshape(1, u)
    out = pl.pallas_call(
        _rgcn_body,
        grid=(ni, nk),
        in_specs=[
            pl.BlockSpec((tk, f), lambda i, k: (jnp.where(i == 0, k, 0), 0)),
            pl.BlockSpec((tm, f), lambda i, k: (i, 0)),
            pl.BlockSpec((tm, tk), lambda i, k: (i, k)),
            pl.BlockSpec((tm, tk), lambda i, k: (i, k)),
            pl.BlockSpec((f, u), lambda i, k: (0, 0)),
            pl.BlockSpec((f, u), lambda i, k: (0, 0)),
            pl.BlockSpec((f, u), lambda i, k: (0, 0)),
            pl.BlockSpec((1, u), lambda i, k: (0, 0)),
        ],
        out_specs=pl.BlockSpec((tm, u), lambda i, k: (i, 0)),
        out_shape=jax.ShapeDtypeStruct((n, u), jnp.float32),
        scratch_shapes=[
            pltpu.VMEM((nk, tk, u), jnp.float32),
            pltpu.VMEM((nk, tk, u), jnp.float32),
            pltpu.VMEM((tm, u), jnp.float32),
        ],
        compiler_params=pltpu.CompilerParams(
            dimension_semantics=("arbitrary", "arbitrary")),
    )(x, x, a0, a1, ws, w0, w1, b2)
    return out


def kernel(features, A_0, A_1, self_kernel, rel_kernel_0, rel_kernel_1, bias):
    x = features[0]
    out = _rgcn(x, A_0, A_1, self_kernel, rel_kernel_0, rel_kernel_1, bias)
    return out[None, ...]
